# baseline (speedup 1.0000x reference)
import functools

import jax
import jax.numpy as jnp
from jax import lax
from jax.experimental import pallas as pl
from jax.experimental.pallas import tpu as pltpu
from jax.experimental.pallas import tpu_sc as plsc

_N = 16384
_D = 1024
_NC = 2
_NS = 16
_NW = _NC * _NS
_ROWS_PER_W = _N // _NW
_CH = 32
_NCHUNK = _ROWS_PER_W // _CH


def _sc_body(x_hbm, out_hbm, buf):
    c = lax.axis_index("c")
    s = lax.axis_index("s")
    wid = s * _NC + c
    row0 = wid * _ROWS_PER_W
    for chunk in range(_NCHUNK):
        start = row0 + chunk * _CH
        pltpu.sync_copy(x_hbm.at[pl.ds(start, _CH)], buf)

        def row_body(r, carry, start=start):
            rowf = (start + r).astype(jnp.float32)
            for j in range(_D // 16):
                sl = pl.ds(j * 16, 16)
                buf[r, sl] = buf[r, sl] * 2.0 + rowf
            return carry

        lax.fori_loop(0, _CH, row_body, 0)
        pltpu.sync_copy(buf, out_hbm.at[pl.ds(start, _CH)])


def kernel(input_tensor):
    mesh = plsc.VectorSubcoreMesh(core_axis_name="c", subcore_axis_name="s")
    kfn = pl.kernel(
        _sc_body,
        out_type=jax.ShapeDtypeStruct((_N, _D), jnp.float32),
        mesh=mesh,
        scratch_types=[pltpu.VMEM((_CH, _D), jnp.float32)],
    )
    return kfn(input_tensor)


# speedup vs baseline: 2.8745x; 2.8745x over previous
import jax
import jax.numpy as jnp
from jax.experimental import pallas as pl
from jax.experimental.pallas import tpu as pltpu

_N = 16384
_D = 1024
_CH = 4096
_NCHUNK = _N // _CH


def _body(x_hbm, o_hbm, b0, b1, insem, outsem):
    bufs = (b0, b1)

    def in_copy(k):
        return pltpu.make_async_copy(
            x_hbm.at[pl.ds(k * _CH, _CH)], bufs[k % 2], insem.at[k % 2])

    def out_copy(k):
        return pltpu.make_async_copy(
            bufs[k % 2], o_hbm.at[pl.ds(k * _CH, _CH)], outsem.at[k % 2])

    in_copy(0).start()
    in_copy(1).start()
    for k in range(_NCHUNK):
        in_copy(k).wait()
        buf = bufs[k % 2]
        row_col = (jax.lax.broadcasted_iota(jnp.int32, (_CH, 1), 0)
                   + k * _CH).astype(jnp.float32)
        buf[...] = buf[...] * 2.0 + row_col
        out_copy(k).start()
        if k + 2 < _NCHUNK:
            out_copy(k).wait()
            in_copy(k + 2).start()
    out_copy(_NCHUNK - 2).wait()
    out_copy(_NCHUNK - 1).wait()


def kernel(input_tensor):
    return pl.pallas_call(
        _body,
        in_specs=[pl.BlockSpec(memory_space=pl.ANY)],
        out_specs=pl.BlockSpec(memory_space=pl.ANY),
        out_shape=jax.ShapeDtypeStruct((_N, _D), input_tensor.dtype),
        scratch_shapes=[
            pltpu.VMEM((_CH, _D), jnp.float32),
            pltpu.VMEM((_CH, _D), jnp.float32),
            pltpu.SemaphoreType.DMA((2,)),
            pltpu.SemaphoreType.DMA((2,)),
        ],
        compiler_params=pltpu.CompilerParams(
            vmem_limit_bytes=64 * 1024 * 1024,
        ),
    )(input_tensor)


# speedup vs baseline: 2.9757x; 1.0352x over previous
import jax
import jax.numpy as jnp
from jax.experimental import pallas as pl
from jax.experimental.pallas import tpu as pltpu

_N = 16384
_D = 1024
_CH = 2048
_NCHUNK = _N // _CH
_NBUF = 4


def _body(x_hbm, o_hbm, *rest):
    bufs = rest[:_NBUF]
    insem, outsem = rest[_NBUF], rest[_NBUF + 1]

    def in_copy(k):
        return pltpu.make_async_copy(
            x_hbm.at[pl.ds(k * _CH, _CH)], bufs[k % _NBUF], insem.at[k % _NBUF])

    def out_copy(k):
        return pltpu.make_async_copy(
            bufs[k % _NBUF], o_hbm.at[pl.ds(k * _CH, _CH)], outsem.at[k % _NBUF])

    for k in range(_NBUF):
        in_copy(k).start()
    for k in range(_NCHUNK):
        in_copy(k).wait()
        buf = bufs[k % _NBUF]
        row_col = (jax.lax.broadcasted_iota(jnp.int32, (_CH, 1), 0)
                   + k * _CH).astype(jnp.float32)
        buf[...] = buf[...] * 2.0 + row_col
        out_copy(k).start()
        if k + _NBUF < _NCHUNK:
            out_copy(k).wait()
            in_copy(k + _NBUF).start()
    for k in range(_NCHUNK - _NBUF, _NCHUNK):
        out_copy(k).wait()


def kernel(input_tensor):
    return pl.pallas_call(
        _body,
        in_specs=[pl.BlockSpec(memory_space=pl.ANY)],
        out_specs=pl.BlockSpec(memory_space=pl.ANY),
        out_shape=jax.ShapeDtypeStruct((_N, _D), input_tensor.dtype),
        scratch_shapes=(
            [pltpu.VMEM((_CH, _D), jnp.float32) for _ in range(_NBUF)]
            + [pltpu.SemaphoreType.DMA((_NBUF,)),
               pltpu.SemaphoreType.DMA((_NBUF,))]
        ),
        compiler_params=pltpu.CompilerParams(
            vmem_limit_bytes=64 * 1024 * 1024,
        ),
    )(input_tensor)
